# baseline (speedup 1.0000x reference)
import functools

import jax
import jax.numpy as jnp
from jax import lax
from jax.experimental import pallas as pl
from jax.experimental.pallas import tpu as pltpu
from jax.experimental.pallas import tpu_sc as plsc

VOCAB = 1000000
EMBED_DIM = 64
BATCH = 16384
HIST = 50

NC, NS = 2, 16
NW = NC * NS
NROWS = VOCAB // 2
NT_FULL = VOCAB // 128
BT = BATCH // 128
BT_PER_W = BT // NW
HP = 56
TILES_PER_W = 200
PITCH = 128

_MESH = plsc.VectorSubcoreMesh(core_axis_name="c", subcore_axis_name="s")
_PARAMS = pltpu.CompilerParams(use_tc_tiling_on_sc=True,
                               needs_layout_passes=False)


def _worker_id():
    return lax.axis_index("s") * NC + lax.axis_index("c")


@functools.partial(
    pl.kernel,
    mesh=_MESH,
    out_type=jax.ShapeDtypeStruct((NROWS, 128), jnp.float32),
    compiler_params=_PARAMS,
    scratch_types=[
        pltpu.VMEM((2, EMBED_DIM, PITCH), jnp.float32),
        pltpu.VMEM((2, EMBED_DIM, 128), jnp.float32),
        pltpu.SemaphoreType.DMA,
        pltpu.SemaphoreType.DMA,
        pltpu.SemaphoreType.DMA,
        pltpu.SemaphoreType.DMA,
    ],
)
def _convert(tableT, tailP, scratch, tin, tout,
             in_sem0, in_sem1, wr_sem0, wr_sem1):
    in_sems = (in_sem0, in_sem1)
    wr_sems = (wr_sem0, wr_sem1)
    w = _worker_id()

    iota = lax.iota(jnp.int32, 16)
    rowm = [16 * m + iota for m in range(4)]

    def in_copy(t, b):
        return pltpu.make_async_copy(
            tableT.at[:, pl.ds(128 * t, 128)],
            tin.at[b, :, pl.ds(0, 128)], in_sems[b])

    def wr_copy(t, b):
        return pltpu.make_async_copy(
            tout.at[b], scratch.at[pl.ds(64 * t, 64)], wr_sems[b])

    def transpose(b):
        def body(j2, _):
            for jj in range(2):
                j = j2 * 2 + jj
                colv = iota * 0 + j
                q = j >> 1
                cb = (j & 1) * EMBED_DIM
                vs = [plsc.load_gather(tin.at[b], [rowm[m], colv])
                      for m in range(4)]
                for m in range(4):
                    tout[b, q, pl.ds(cb + 16 * m, 16)] = vs[m]
            return 0
        lax.fori_loop(0, 64, body, 0)

    in_copy(w, 0).start()
    in_copy(w + 32, 1).start()

    def body(g, _):
        for b in range(2):
            ti = 2 * g + b
            t = w + 32 * ti
            in_copy(t, b).wait()

            @pl.when(g >= 1)
            def _():
                wr_copy(t - 64, b).wait()

            transpose(b)

            @pl.when(w + 32 * (ti + 2) < NT_FULL)
            def _():
                in_copy(t + 64, b).start()

            wr_copy(t, b).start()
        return 0

    lax.fori_loop(0, 122, body, 0)

    t_last = w + 32 * 244

    @pl.when(w < 4)
    def _():
        in_copy(t_last, 0).wait()
        wr_copy(t_last - 64, 0).wait()
        transpose(0)
        wr_copy(t_last, 0).start()

    wr_copy(0, 1).wait()
    wr_copy(0, 0).wait()

    @pl.when(w == NW - 1)
    def _():
        pltpu.sync_copy(tailP, tin.at[0, :, pl.ds(0, 128)])
        transpose(0)
        pltpu.sync_copy(tout.at[0], scratch.at[pl.ds(NROWS - 64, 64)])


@functools.partial(
    pl.kernel,
    mesh=_MESH,
    out_type=jax.ShapeDtypeStruct((HIST, 8, BT, 8, 128), jnp.float32),
    compiler_params=_PARAMS,
    scratch_types=[
        pltpu.VMEM((BT_PER_W, HP, 128), jnp.int32),
        pltpu.VMEM((2, 128), jnp.int32),
        pltpu.VMEM((2, 128, PITCH), jnp.float32),
        pltpu.VMEM((2, 8, 1, 8, 128), jnp.float32),
        pltpu.SemaphoreType.DMA,
        pltpu.SemaphoreType.DMA,
        pltpu.SemaphoreType.DMA,
        pltpu.SemaphoreType.DMA,
    ],
)
def _gather(scratch, gidx, out, idxv, rowbuf, pairs, ostg,
            gat_sem0, gat_sem1, wr_sem0, wr_sem1):
    gat_sems = (gat_sem0, gat_sem1)
    wr_sems = (wr_sem0, wr_sem1)
    w = _worker_id()
    bt0 = w * BT_PER_W

    iota = lax.iota(jnp.int32, 16)
    rowv = [16 * c + iota for c in range(8)]

    pltpu.sync_copy(gidx.at[pl.ds(bt0, BT_PER_W)], idxv)

    def prep_rows(ht, b):
        j = ht // HIST
        h = ht % HIST
        for c in range(8):
            rowbuf[b, pl.ds(16 * c, 16)] = idxv[j, h, pl.ds(16 * c, 16)] >> 1

    def gat_copy(ht, b):
        return pltpu.make_async_copy(
            scratch.at[rowbuf.at[b]],
            pairs.at[b, :, pl.ds(0, 128)], gat_sems[b])

    def wr_copy(ht, b):
        j = ht // HIST
        h = ht % HIST
        return pltpu.make_async_copy(
            ostg.at[b], out.at[pl.ds(h, 1), :, pl.ds(bt0 + j, 1)].at[0],
            wr_sems[b])

    prep_rows(0, 0)
    gat_copy(0, 0).start()

    def body(g, _):
        for b in range(2):
            ht = 2 * g + b
            j = ht // HIST
            h = ht % HIST
            gat_copy(ht, b).wait()

            @pl.when(ht + 1 < TILES_PER_W)
            def _():
                prep_rows(ht + 1, 1 - b)
                gat_copy(ht + 1, 1 - b).start()

            @pl.when(g >= 1)
            def _():
                wr_copy(ht - 2, b).wait()

            def tbody(c, _):
                c16 = 16 * c
                rowc = c16 + iota
                pv = (idxv[j, h, pl.ds(c16, 16)] & 1) << 6
                for s in range(8):
                    colvs = [pv + (8 * s + r) for r in range(8)]
                    gs = [plsc.load_gather(pairs.at[b], [rowc, colvs[r]])
                          for r in range(8)]
                    for r in range(8):
                        ostg[b, s, 0, r, pl.ds(c16, 16)] = gs[r]
                return 0

            lax.fori_loop(0, 8, tbody, 0)
            wr_copy(ht, b).start()
        return 0

    lax.fori_loop(0, TILES_PER_W // 2, body, 0)

    wr_copy(TILES_PER_W - 2, 0).wait()
    wr_copy(TILES_PER_W - 1, 1).wait()


def kernel(inputs, table):
    tableT = table.T
    tailP = table[VOCAB - 128:].T
    idxT = inputs.T.reshape(HIST, BT, 128).transpose(1, 0, 2)
    gidx = jnp.pad(idxT, ((0, 0), (0, HP - HIST), (0, 0)))
    scratch = table.reshape(NROWS, 128)
    out5d = _gather(scratch, gidx)
    return out5d.transpose(2, 4, 0, 1, 3).reshape(BATCH, HIST, EMBED_DIM)


# speedup vs baseline: 1.0000x; 1.0000x over previous
import functools

import jax
import jax.numpy as jnp
from jax import lax
from jax.experimental import pallas as pl
from jax.experimental.pallas import tpu as pltpu
from jax.experimental.pallas import tpu_sc as plsc

VOCAB = 1000000
EMBED_DIM = 64
BATCH = 16384
HIST = 50

NC, NS = 2, 16
NW = NC * NS
NROWS = VOCAB // 2
NT_FULL = VOCAB // 128
BT = BATCH // 128
BT_PER_W = BT // NW
HP = 56
TILES_PER_W = 200
PITCH = 128

_MESH = plsc.VectorSubcoreMesh(core_axis_name="c", subcore_axis_name="s")
_PARAMS = pltpu.CompilerParams(use_tc_tiling_on_sc=True,
                               needs_layout_passes=False)


def _worker_id():
    return lax.axis_index("s") * NC + lax.axis_index("c")


@functools.partial(
    pl.kernel,
    mesh=_MESH,
    out_type=jax.ShapeDtypeStruct((NROWS, 128), jnp.float32),
    compiler_params=_PARAMS,
    scratch_types=[
        pltpu.VMEM((2, EMBED_DIM, PITCH), jnp.float32),
        pltpu.VMEM((2, EMBED_DIM, 128), jnp.float32),
        pltpu.SemaphoreType.DMA,
        pltpu.SemaphoreType.DMA,
        pltpu.SemaphoreType.DMA,
        pltpu.SemaphoreType.DMA,
    ],
)
def _convert(tableT, tailP, scratch, tin, tout,
             in_sem0, in_sem1, wr_sem0, wr_sem1):
    in_sems = (in_sem0, in_sem1)
    wr_sems = (wr_sem0, wr_sem1)
    w = _worker_id()

    iota = lax.iota(jnp.int32, 16)
    rowm = [16 * m + iota for m in range(4)]

    def in_copy(t, b):
        return pltpu.make_async_copy(
            tableT.at[:, pl.ds(128 * t, 128)],
            tin.at[b, :, pl.ds(0, 128)], in_sems[b])

    def wr_copy(t, b):
        return pltpu.make_async_copy(
            tout.at[b], scratch.at[pl.ds(64 * t, 64)], wr_sems[b])

    def transpose(b):
        def body(j2, _):
            for jj in range(2):
                j = j2 * 2 + jj
                colv = iota * 0 + j
                q = j >> 1
                cb = (j & 1) * EMBED_DIM
                vs = [plsc.load_gather(tin.at[b], [rowm[m], colv])
                      for m in range(4)]
                for m in range(4):
                    tout[b, q, pl.ds(cb + 16 * m, 16)] = vs[m]
            return 0
        lax.fori_loop(0, 64, body, 0)

    in_copy(w, 0).start()
    in_copy(w + 32, 1).start()

    def body(g, _):
        for b in range(2):
            ti = 2 * g + b
            t = w + 32 * ti
            in_copy(t, b).wait()

            @pl.when(g >= 1)
            def _():
                wr_copy(t - 64, b).wait()

            transpose(b)

            @pl.when(w + 32 * (ti + 2) < NT_FULL)
            def _():
                in_copy(t + 64, b).start()

            wr_copy(t, b).start()
        return 0

    lax.fori_loop(0, 122, body, 0)

    t_last = w + 32 * 244

    @pl.when(w < 4)
    def _():
        in_copy(t_last, 0).wait()
        wr_copy(t_last - 64, 0).wait()
        transpose(0)
        wr_copy(t_last, 0).start()

    wr_copy(0, 1).wait()
    wr_copy(0, 0).wait()

    @pl.when(w == NW - 1)
    def _():
        pltpu.sync_copy(tailP, tin.at[0, :, pl.ds(0, 128)])
        transpose(0)
        pltpu.sync_copy(tout.at[0], scratch.at[pl.ds(NROWS - 64, 64)])


@functools.partial(
    pl.kernel,
    mesh=_MESH,
    out_type=jax.ShapeDtypeStruct((HIST, 8, BT, 8, 128), jnp.float32),
    compiler_params=_PARAMS,
    scratch_types=[
        pltpu.VMEM((BT_PER_W, HP, 128), jnp.int32),
        pltpu.VMEM((2, 128), jnp.int32),
        pltpu.VMEM((2, 128, PITCH), jnp.float32),
        pltpu.VMEM((2, 8, 1, 8, 128), jnp.float32),
        pltpu.SemaphoreType.DMA,
        pltpu.SemaphoreType.DMA,
        pltpu.SemaphoreType.DMA,
        pltpu.SemaphoreType.DMA,
    ],
)
def _gather(scratch, gidx, out, idxv, rowbuf, pairs, ostg,
            gat_sem0, gat_sem1, wr_sem0, wr_sem1):
    gat_sems = (gat_sem0, gat_sem1)
    wr_sems = (wr_sem0, wr_sem1)
    w = _worker_id()
    bt0 = w * BT_PER_W

    iota = lax.iota(jnp.int32, 16)
    rowv = [16 * c + iota for c in range(8)]

    pltpu.sync_copy(gidx.at[pl.ds(bt0, BT_PER_W)], idxv)

    def prep_rows(ht, b):
        j = ht // HIST
        h = ht % HIST
        for c in range(8):
            rowbuf[b, pl.ds(16 * c, 16)] = idxv[j, h, pl.ds(16 * c, 16)] >> 1

    def gat_copy(ht, b):
        return pltpu.make_async_copy(
            scratch.at[rowbuf.at[b]],
            pairs.at[b, :, pl.ds(0, 128)], gat_sems[b])

    def wr_copy(ht, b):
        j = ht // HIST
        h = ht % HIST
        return pltpu.make_async_copy(
            ostg.at[b], out.at[pl.ds(h, 1), :, pl.ds(bt0 + j, 1)].at[0],
            wr_sems[b])

    prep_rows(0, 0)
    gat_copy(0, 0).start()

    def body(g, _):
        for b in range(2):
            ht = 2 * g + b
            j = ht // HIST
            h = ht % HIST
            gat_copy(ht, b).wait()

            @pl.when(ht + 1 < TILES_PER_W)
            def _():
                prep_rows(ht + 1, 1 - b)
                gat_copy(ht + 1, 1 - b).start()

            @pl.when(g >= 1)
            def _():
                wr_copy(ht - 2, b).wait()

            def tbody(c2, _):
                for cc in range(2):
                    c16 = 32 * c2 + 16 * cc
                    rowc = c16 + iota
                    pv = (idxv[j, h, pl.ds(c16, 16)] & 1) << 6
                    for s in range(8):
                        colvs = [pv + (8 * s + r) for r in range(8)]
                        gs = [plsc.load_gather(pairs.at[b], [rowc, colvs[r]])
                              for r in range(8)]
                        for r in range(8):
                            ostg[b, s, 0, r, pl.ds(c16, 16)] = gs[r]
                return 0

            lax.fori_loop(0, 4, tbody, 0)
            wr_copy(ht, b).start()
        return 0

    lax.fori_loop(0, TILES_PER_W // 2, body, 0)

    wr_copy(TILES_PER_W - 2, 0).wait()
    wr_copy(TILES_PER_W - 1, 1).wait()


def kernel(inputs, table):
    tableT = table.T
    tailP = table[VOCAB - 128:].T
    idxT = inputs.T.reshape(HIST, BT, 128).transpose(1, 0, 2)
    gidx = jnp.pad(idxT, ((0, 0), (0, HP - HIST), (0, 0)))
    scratch = table.reshape(NROWS, 128)
    out5d = _gather(scratch, gidx)
    return out5d.transpose(2, 4, 0, 1, 3).reshape(BATCH, HIST, EMBED_DIM)


# speedup vs baseline: 1.0631x; 1.0631x over previous
import functools

import jax
import jax.numpy as jnp
from jax import lax
from jax.experimental import pallas as pl
from jax.experimental.pallas import tpu as pltpu
from jax.experimental.pallas import tpu_sc as plsc

VOCAB = 1000000
EMBED_DIM = 64
BATCH = 16384
HIST = 50

B = BATCH * HIST
NC, NS = 2, 16
NW = NC * NS
B_PER_W = B // NW
IDX_MINOR = 128
N_SUB = 4
CHUNK = N_SUB * IDX_MINOR
N_ITERS = B_PER_W // CHUNK
IDX_ROWS_PER_CHUNK = CHUNK // IDX_MINOR


def _make_kernel():
    mesh = plsc.VectorSubcoreMesh(core_axis_name="c", subcore_axis_name="s")

    @functools.partial(
        pl.kernel,
        mesh=mesh,
        out_type=jax.ShapeDtypeStruct((B, EMBED_DIM), jnp.float32),
        compiler_params=pltpu.CompilerParams(use_tc_tiling_on_sc=False),
        scratch_types=[
            pltpu.VMEM((2, N_SUB, IDX_MINOR), jnp.int32),
            pltpu.VMEM((2, CHUNK, EMBED_DIM), jnp.float32),
            pltpu.SemaphoreType.DMA,
            pltpu.SemaphoreType.DMA,
            pltpu.SemaphoreType.DMA,
            pltpu.SemaphoreType.DMA,
            pltpu.SemaphoreType.DMA,
            pltpu.SemaphoreType.DMA,
        ],
    )
    def k(idx_hbm, table_hbm, out_hbm, idx_v, rows_v,
          idx_sem0, idx_sem1, gat_sem0, gat_sem1, out_sem0, out_sem1):
        idx_sems = (idx_sem0, idx_sem1)
        gat_sems = (gat_sem0, gat_sem1)
        out_sems = (out_sem0, out_sem1)

        wid = lax.axis_index("s") * NC + lax.axis_index("c")
        base = wid * B_PER_W
        base128 = wid * (B_PER_W // IDX_MINOR)

        def idx_copy(i, b):
            return pltpu.make_async_copy(
                idx_hbm.at[pl.ds(base128 + i * N_SUB, N_SUB)],
                idx_v.at[b],
                idx_sems[b],
            )

        def fire_gathers(b):
            for j in range(N_SUB):
                pltpu.async_copy(
                    table_hbm.at[idx_v.at[b].at[j]],
                    rows_v.at[b].at[pl.ds(j * IDX_MINOR, IDX_MINOR)],
                    gat_sems[b],
                )

        def wait_gathers(b):
            for j in range(N_SUB):
                pltpu.make_async_copy(
                    table_hbm.at[idx_v.at[b].at[j]],
                    rows_v.at[b].at[pl.ds(j * IDX_MINOR, IDX_MINOR)],
                    gat_sems[b],
                ).wait()

        def store_copy(i, b):
            return pltpu.make_async_copy(
                rows_v.at[b],
                out_hbm.at[pl.ds(base + i * CHUNK, CHUNK)],
                out_sems[b],
            )

        idx_copy(0, 0).start()
        idx_copy(1, 1).start()
        idx_copy(0, 0).wait()
        fire_gathers(0)

        def body(g, _):
            for b in range(2):
                i = 2 * g + b
                wait_gathers(b)
                @pl.when(i + 2 < N_ITERS)
                def _():
                    idx_copy(i + 2, b).start()
                store_copy(i, b).start()
                @pl.when(i + 1 < N_ITERS)
                def _():
                    idx_copy(i + 1, 1 - b).wait()

                    @pl.when(i >= 1)
                    def _():
                        store_copy(i - 1, 1 - b).wait()

                    fire_gathers(1 - b)
            return 0

        lax.fori_loop(0, N_ITERS // 2, body, 0)

        store_copy(N_ITERS - 2, 0).wait()
        store_copy(N_ITERS - 1, 1).wait()

    return k


_gather = _make_kernel()


def kernel(inputs, table):
    idx2d = inputs.reshape(B // IDX_MINOR, IDX_MINOR)
    out = _gather(idx2d, table)
    return out.reshape(BATCH, HIST, EMBED_DIM)


# speedup vs baseline: 1.0662x; 1.0029x over previous
import functools

import jax
import jax.numpy as jnp
from jax import lax
from jax.experimental import pallas as pl
from jax.experimental.pallas import tpu as pltpu
from jax.experimental.pallas import tpu_sc as plsc

VOCAB = 1000000
EMBED_DIM = 64
BATCH = 16384
HIST = 50

B = BATCH * HIST
NC, NS = 2, 16
NW = NC * NS
B_PER_W = B // NW
IDX_MINOR = 128
N_SUB = 4
CHUNK = N_SUB * IDX_MINOR
N_ITERS = B_PER_W // CHUNK
IDX_ROWS_PER_CHUNK = CHUNK // IDX_MINOR


def _make_kernel():
    mesh = plsc.VectorSubcoreMesh(core_axis_name="c", subcore_axis_name="s")

    @functools.partial(
        pl.kernel,
        mesh=mesh,
        out_type=jax.ShapeDtypeStruct((B, EMBED_DIM), jnp.float32),
        compiler_params=pltpu.CompilerParams(use_tc_tiling_on_sc=False),
        scratch_types=[
            pltpu.VMEM((2, N_SUB, IDX_MINOR), jnp.int32),
            pltpu.VMEM((2, CHUNK, EMBED_DIM), jnp.float32),
            pltpu.SemaphoreType.DMA,
            pltpu.SemaphoreType.DMA,
            pltpu.SemaphoreType.DMA,
            pltpu.SemaphoreType.DMA,
            pltpu.SemaphoreType.DMA,
            pltpu.SemaphoreType.DMA,
        ],
    )
    def k(idx_hbm, table_hbm, out_hbm, idx_v, rows_v,
          idx_sem0, idx_sem1, gat_sem0, gat_sem1, out_sem0, out_sem1):
        idx_sems = (idx_sem0, idx_sem1)
        gat_sems = (gat_sem0, gat_sem1)
        out_sems = (out_sem0, out_sem1)

        wid = lax.axis_index("s") * NC + lax.axis_index("c")
        base = wid * B_PER_W
        base128 = wid * (B_PER_W // IDX_MINOR)

        def idx_copy(i, b):
            return pltpu.make_async_copy(
                idx_hbm.at[pl.ds(base128 + i * N_SUB, N_SUB)],
                idx_v.at[b],
                idx_sems[b],
            )

        def fire_gathers(b):
            for j in range(N_SUB):
                pltpu.async_copy(
                    table_hbm.at[idx_v.at[b].at[j]],
                    rows_v.at[b].at[pl.ds(j * IDX_MINOR, IDX_MINOR)],
                    gat_sems[b],
                )

        def wait_gathers(b):
            for j in range(N_SUB):
                pltpu.make_async_copy(
                    table_hbm.at[idx_v.at[b].at[j]],
                    rows_v.at[b].at[pl.ds(j * IDX_MINOR, IDX_MINOR)],
                    gat_sems[b],
                ).wait()

        def store_copy(i, b):
            return pltpu.make_async_copy(
                rows_v.at[b],
                out_hbm.at[pl.ds(base + i * CHUNK, CHUNK)],
                out_sems[b],
            )

        idx_copy(0, 0).start()
        idx_copy(1, 1).start()
        idx_copy(0, 0).wait()
        fire_gathers(0)

        def body(g, _):
            for b in range(2):
                i = 2 * g + b
                @pl.when(i + 1 < N_ITERS)
                def _():
                    idx_copy(i + 1, 1 - b).wait()

                    @pl.when(i >= 1)
                    def _():
                        store_copy(i - 1, 1 - b).wait()

                    fire_gathers(1 - b)

                wait_gathers(b)
                @pl.when(i + 2 < N_ITERS)
                def _():
                    idx_copy(i + 2, b).start()
                store_copy(i, b).start()
            return 0

        lax.fori_loop(0, N_ITERS // 2, body, 0)

        store_copy(N_ITERS - 2, 0).wait()
        store_copy(N_ITERS - 1, 1).wait()

    return k


_gather = _make_kernel()


def kernel(inputs, table):
    idx2d = inputs.reshape(B // IDX_MINOR, IDX_MINOR)
    out = _gather(idx2d, table)
    return out.reshape(BATCH, HIST, EMBED_DIM)
